# Initial kernel scaffold; baseline (speedup 1.0000x reference)
import jax
import jax.numpy as jnp
from jax.experimental import pallas as pl


def kernel(embeddings):
    raise NotImplementedError("write your pallas kernel here")


# speedup vs baseline: 7.6114x; 7.6114x over previous
import functools

import numpy as np
import jax
import jax.numpy as jnp
from jax import lax
from jax.experimental import pallas as pl
from jax.experimental.pallas import tpu as pltpu
from jax.experimental.pallas import tpu_sc as plsc

DIM_ = 64
ROWS_ = 800
N_WORDS_ = 80
LANES_ = 16
TILES_ = 32
WSLAB_ = 49
SPW_ = 0.1 / (DIM_ - 1)
SCALE_ = 0.5 / N_WORDS_


def _build_tables():
    triples = []
    for d in range(8):
        triples += [(d * 100 + j, d, 0) for j in range(4)]
    for d in range(8):
        triples += [(d * 100 + 10 + j, d, 1) for j in range(4)]
    for d in range(8):
        triples += [(d * 100 + 20 + j, d, 2) for j in range(2)]
    assert len(triples) == N_WORDS_
    idx = np.zeros((TILES_, LANES_), np.int32)
    wts = np.zeros((TILES_, WSLAB_, LANES_), np.float32)
    for i, (w, d, c) in enumerate(triples):
        t, l = divmod(i, LANES_)
        idx[t, l] = w
        wts[t, 16 * c + l, d] = 1.0
        if c in (0, 1):
            wts[t, 48, :] = SPW_
    return idx, wts


_IDX, _WTS = _build_tables()


def _body(table, idxa, wtsa, out,
          idx_v, wts_v, rows_v, tv_v, shl_v, sh, sem):
    c = lax.axis_index("c")
    s = lax.axis_index("s")
    wid = c * 16 + s

    pltpu.sync_copy(idxa.at[wid], idx_v)
    gather = pltpu.async_copy(table.at[idx_v], rows_v, sem)
    pltpu.sync_copy(wtsa.at[wid], wts_v)
    gather.wait()

    zero = jnp.zeros((16,), jnp.float32)
    spw = wts_v[48, 0:16]
    acc = zero
    f = zero
    for j in range(16):
        s0 = rows_v[j, 0:16]
        a0 = jnp.abs(s0)
        fp = jnp.where(s0 <= 0.0, a0 + 0.1, -0.1 * s0) - SPW_ * a0
        fn = jnp.where(s0 >= 0.0, a0 + 0.1, 0.1 * s0) - SPW_ * a0
        fu = 2.0 * a0
        f = (f + fp * wts_v[j, 0:16] + fn * wts_v[16 + j, 0:16]
             + fu * wts_v[32 + j, 0:16])
        acc = acc + a0
        for k in range(1, 4):
            acc = acc + jnp.abs(rows_v[j, 16 * k:16 * (k + 1)])

    tv_v[...] = f + acc * spw
    pltpu.sync_copy(tv_v, sh.at[s])
    plsc.subcore_barrier()

    @pl.when(jnp.logical_and(c == 0, s == 0))
    def _():
        pltpu.sync_copy(sh, shl_v)
        g = jnp.zeros((16,), jnp.float32)
        for j in range(16):
            g = g + shl_v[j, 0:16]
        total = jnp.float32(0.0)
        for j in range(16):
            total = total + g[j]
        total = total * SCALE_
        tv_v[...] = jnp.full((16,), total, jnp.float32)
        pltpu.sync_copy(tv_v, out)


_sc_call = functools.partial(
    pl.kernel,
    mesh=plsc.VectorSubcoreMesh(core_axis_name="c", subcore_axis_name="s"),
    out_type=jax.ShapeDtypeStruct((16,), jnp.float32),
    compiler_params=pltpu.CompilerParams(use_tc_tiling_on_sc=False),
    scratch_types=[
        pltpu.VMEM((LANES_,), jnp.int32),
        pltpu.VMEM((WSLAB_, LANES_), jnp.float32),
        pltpu.VMEM((LANES_, DIM_), jnp.float32),
        pltpu.VMEM((LANES_,), jnp.float32),
        pltpu.VMEM((LANES_, LANES_), jnp.float32),
        pltpu.VMEM_SHARED((LANES_, LANES_), jnp.float32),
        pltpu.SemaphoreType.DMA,
    ],
)(_body)


@jax.jit
def kernel(embeddings):
    out = _sc_call(embeddings[:ROWS_],
                   jnp.asarray(_IDX), jnp.asarray(_WTS))
    return out[0]


# speedup vs baseline: 8.4757x; 1.1136x over previous
import functools

import jax
import jax.numpy as jnp
from jax import lax
from jax.experimental import pallas as pl
from jax.experimental.pallas import tpu as pltpu
from jax.experimental.pallas import tpu_sc as plsc

DIM_ = 64
ROWS_ = 800
N_WORDS_ = 80
LANES_ = 16
SPW_ = 0.1 / (DIM_ - 1)
SCALE_ = 0.5 / N_WORDS_


def _body(table, out, idx_v, rows_v, tv_v, shl_v, sh, sem):
    c = lax.axis_index("c")
    s = lax.axis_index("s")
    wid = c * 16 + s

    lanes = lax.iota(jnp.int32, 16)
    g = wid * 16 + lanes
    d = jnp.where(g < 32, g >> 2,
                  jnp.where(g < 64, (g - 32) >> 2, (g - 64) >> 1))
    off = jnp.where(g < 32, g & 3,
                    jnp.where(g < 64, 10 + ((g - 32) & 3),
                              20 + ((g - 64) & 1)))
    word = jnp.where(g < 80, d * 100 + off, 0)
    idx_v[...] = word
    gather = pltpu.async_copy(table.at[idx_v], rows_v, sem)

    fzero = jnp.zeros((16,), jnp.float32)
    fone = jnp.ones((16,), jnp.float32)
    spw = jnp.where(g < 64, jnp.float32(SPW_), jnp.float32(0.0))
    onehots, wps, wns, wus = [], [], [], []
    base = wid * 16
    for j in range(16):
        gj = base + j
        onehots.append(jnp.where(lanes == d[j], fone, fzero))
        wps.append(jnp.where(gj < 32, jnp.float32(1.0), jnp.float32(0.0)))
        wns.append(jnp.where(jnp.logical_and(gj >= 32, gj < 64),
                             jnp.float32(1.0), jnp.float32(0.0)))
        wus.append(jnp.where(jnp.logical_and(gj >= 64, gj < 80),
                             jnp.float32(1.0), jnp.float32(0.0)))
    gather.wait()

    acc = fzero
    f = fzero
    for j in range(16):
        s0 = rows_v[j, 0:16]
        a0 = jnp.abs(s0)
        fp = jnp.where(s0 <= 0.0, a0 + 0.1, -0.1 * s0) - SPW_ * a0
        fn = jnp.where(s0 >= 0.0, a0 + 0.1, 0.1 * s0) - SPW_ * a0
        fu = 2.0 * a0
        f = f + (fp * wps[j] + fn * wns[j] + fu * wus[j]) * onehots[j]
        acc = acc + a0
        for k in range(1, 4):
            acc = acc + jnp.abs(rows_v[j, 16 * k:16 * (k + 1)])

    tv_v[...] = f + acc * spw
    pltpu.sync_copy(tv_v, sh.at[s])
    plsc.subcore_barrier()

    @pl.when(jnp.logical_and(c == 0, s == 0))
    def _():
        pltpu.sync_copy(sh, shl_v)
        gsum = fzero
        for j in range(16):
            gsum = gsum + shl_v[j, 0:16]
        total = jnp.float32(0.0)
        for j in range(16):
            total = total + gsum[j]
        total = total * SCALE_
        tv_v[...] = jnp.full((16,), total, jnp.float32)
        pltpu.sync_copy(tv_v.at[0:1], out)


_sc_call = functools.partial(
    pl.kernel,
    mesh=plsc.VectorSubcoreMesh(core_axis_name="c", subcore_axis_name="s"),
    out_type=jax.ShapeDtypeStruct((1,), jnp.float32),
    compiler_params=pltpu.CompilerParams(use_tc_tiling_on_sc=False),
    scratch_types=[
        pltpu.VMEM((LANES_,), jnp.int32),
        pltpu.VMEM((LANES_, DIM_), jnp.float32),
        pltpu.VMEM((LANES_,), jnp.float32),
        pltpu.VMEM((LANES_, LANES_), jnp.float32),
        pltpu.VMEM_SHARED((LANES_, LANES_), jnp.float32),
        pltpu.SemaphoreType.DMA,
    ],
)(_body)


@jax.jit
def kernel(embeddings):
    out = _sc_call(embeddings[:ROWS_])
    return jnp.reshape(out, ())


# speedup vs baseline: 10.6379x; 1.2551x over previous
import functools

import jax
import jax.numpy as jnp
from jax import lax
from jax.experimental import pallas as pl
from jax.experimental.pallas import tpu as pltpu
from jax.experimental.pallas import tpu_sc as plsc

DIM_ = 64
ROWS_ = 800
N_WORDS_ = 80
LANES_ = 16
SPW_ = 0.1 / (DIM_ - 1)
SCALE_ = 0.5 / N_WORDS_


def _body(table, out, idx_v, rows_v, tv_v, shl_v, sh, sem):
    c = lax.axis_index("c")
    s = lax.axis_index("s")
    wid = c * 16 + s

    lanes = lax.iota(jnp.int32, 16)
    g = wid * 16 + lanes
    d = jnp.where(g < 32, g >> 2,
                  jnp.where(g < 64, (g - 32) >> 2, (g - 64) >> 1))
    off = jnp.where(g < 32, g & 3,
                    jnp.where(g < 64, 10 + ((g - 32) & 3),
                              20 + ((g - 64) & 1)))
    word = jnp.where(g < 80, d * 100 + off, 0)
    idx_v[...] = word
    gather = pltpu.async_copy(table.at[idx_v], rows_v, sem)

    fzero = jnp.zeros((16,), jnp.float32)
    fone = jnp.ones((16,), jnp.float32)
    spw = jnp.where(g < 64, jnp.float32(SPW_), jnp.float32(0.0))
    onehots, wps, wns, wus = [], [], [], []
    base = wid * 16
    for j in range(16):
        gj = base + j
        onehots.append(jnp.where(lanes == d[j], fone, fzero))
        wps.append(jnp.where(gj < 32, jnp.float32(1.0), jnp.float32(0.0)))
        wns.append(jnp.where(jnp.logical_and(gj >= 32, gj < 64),
                             jnp.float32(1.0), jnp.float32(0.0)))
        wus.append(jnp.where(jnp.logical_and(gj >= 64, gj < 80),
                             jnp.float32(1.0), jnp.float32(0.0)))
    gather.wait()

    acc = fzero
    f = fzero
    for j in range(16):
        s0 = rows_v[j, 0:16]
        a0 = jnp.abs(s0)
        fp = jnp.where(s0 <= 0.0, a0 + 0.1, -0.1 * s0) - SPW_ * a0
        fn = jnp.where(s0 >= 0.0, a0 + 0.1, 0.1 * s0) - SPW_ * a0
        fu = 2.0 * a0
        f = f + (fp * wps[j] + fn * wns[j] + fu * wus[j]) * onehots[j]
        acc = acc + a0
        for k in range(1, 4):
            acc = acc + jnp.abs(rows_v[j, 16 * k:16 * (k + 1)])

    tv_v[...] = f + acc * spw
    pltpu.sync_copy(tv_v, sh.at[s])
    plsc.subcore_barrier()

    @pl.when(jnp.logical_and(c == 0, s == 0))
    def _():
        pltpu.sync_copy(sh, shl_v)
        gsum = fzero
        for j in range(16):
            gsum = gsum + shl_v[j, 0:16]
        total = jnp.float32(0.0)
        for j in range(16):
            total = total + gsum[j]
        total = total * SCALE_
        tv_v[...] = jnp.full((16,), total, jnp.float32)
        pltpu.sync_copy(tv_v.at[0:1], out)


_sc_call = functools.partial(
    pl.kernel,
    mesh=plsc.VectorSubcoreMesh(core_axis_name="c", subcore_axis_name="s",
                                num_cores=1),
    out_type=jax.ShapeDtypeStruct((1,), jnp.float32),
    compiler_params=pltpu.CompilerParams(use_tc_tiling_on_sc=False),
    scratch_types=[
        pltpu.VMEM((LANES_,), jnp.int32),
        pltpu.VMEM((LANES_, DIM_), jnp.float32),
        pltpu.VMEM((LANES_,), jnp.float32),
        pltpu.VMEM((LANES_, LANES_), jnp.float32),
        pltpu.VMEM_SHARED((LANES_, LANES_), jnp.float32),
        pltpu.SemaphoreType.DMA,
    ],
)(_body)


@jax.jit
def kernel(embeddings):
    out = _sc_call(embeddings[:ROWS_])
    return jnp.reshape(out, ())


# speedup vs baseline: 11.8023x; 1.1095x over previous
import functools

import jax
import jax.numpy as jnp
from jax import lax
from jax.experimental import pallas as pl
from jax.experimental.pallas import tpu as pltpu
from jax.experimental.pallas import tpu_sc as plsc

DIM_ = 64
ROWS_ = 800
N_WORDS_ = 80
LANES_ = 16
SPW_ = 0.1 / (DIM_ - 1)
SCALE_ = 0.5 / N_WORDS_


def _word_meta(g):
    if g < 32:
        return (g // 4) * 100 + g % 4, g // 4, 0
    if g < 64:
        h = g - 32
        return (h // 4) * 100 + 10 + h % 4, h // 4, 1
    h = g - 64
    return (h // 2) * 100 + 20 + h % 2, h // 2, 2


def _body(table, out, idx_v, rows_v, tv_v, sem):
    lanes = lax.iota(jnp.int32, 16)
    for t in range(5):
        g = t * 16 + lanes
        d = jnp.where(g < 32, g >> 2,
                      jnp.where(g < 64, (g - 32) >> 2, (g - 64) >> 1))
        off = jnp.where(g < 32, g & 3,
                        jnp.where(g < 64, 10 + ((g - 32) & 3),
                                  20 + ((g - 64) & 1)))
        idx_v[16 * t:16 * (t + 1)] = d * 100 + off
    gather = pltpu.async_copy(table.at[idx_v], rows_v, sem)

    fzero = jnp.zeros((16,), jnp.float32)
    fone = jnp.ones((16,), jnp.float32)
    onehots = [jnp.where(lanes == _word_meta(g)[1], fone, fzero)
               for g in range(N_WORDS_)]
    gather.wait()

    acc = fzero
    f = fzero
    for g in range(N_WORDS_):
        _, _, cls = _word_meta(g)
        s0 = rows_v[g, 0:16]
        a0 = jnp.abs(s0)
        if cls == 0:
            fg = jnp.where(s0 <= 0.0, a0 + 0.1, -0.1 * s0) - SPW_ * a0
        elif cls == 1:
            fg = jnp.where(s0 >= 0.0, a0 + 0.1, 0.1 * s0) - SPW_ * a0
        else:
            fg = 2.0 * a0
        f = f + fg * onehots[g]
        if cls != 2:
            acc = acc + a0
            for k in range(1, 4):
                acc = acc + jnp.abs(rows_v[g, 16 * k:16 * (k + 1)])

    total_vec = f + SPW_ * acc
    total = jnp.float32(0.0)
    for j in range(16):
        total = total + total_vec[j]
    total = total * SCALE_
    tv_v[...] = jnp.full((16,), total, jnp.float32)
    pltpu.sync_copy(tv_v.at[0:1], out)


_sc_call = functools.partial(
    pl.kernel,
    mesh=plsc.VectorSubcoreMesh(core_axis_name="c", subcore_axis_name="s",
                                num_cores=1, num_subcores=1),
    out_type=jax.ShapeDtypeStruct((1,), jnp.float32),
    compiler_params=pltpu.CompilerParams(use_tc_tiling_on_sc=False),
    scratch_types=[
        pltpu.VMEM((N_WORDS_,), jnp.int32),
        pltpu.VMEM((N_WORDS_, DIM_), jnp.float32),
        pltpu.VMEM((LANES_,), jnp.float32),
        pltpu.SemaphoreType.DMA,
    ],
)(_body)


@jax.jit
def kernel(embeddings):
    out = _sc_call(embeddings[:ROWS_])
    return jnp.reshape(out, ())
